# baseline (speedup 1.0000x reference)
import numpy as np
import jax
import jax.numpy as jnp
from jax import lax
from jax.experimental import pallas as pl
from jax.experimental.pallas import tpu as pltpu
from jax.experimental.pallas import tpu_sc as plsc

N_LEVELS = 16
F = 2
LOG2_T = 19
T = 1 << LOG2_T
MASK = T - 1
BASE_RES = 16
PLS = 1.39

LANES = 16
NC, NS = 2, 16
NW = NC * NS
PB = 128
G = PB // LANES

_RES = [float(np.floor(BASE_RES * (PLS ** l))) for l in range(N_LEVELS)]
_PRIMES_U = [1, 2654435761, 805459861, 3674653429]
_PRIMES = [p - (1 << 32) if p >= (1 << 31) else p for p in _PRIMES_U]

_GDN = lax.GatherDimensionNumbers(
    offset_dims=(), collapsed_slice_dims=(0,), start_index_map=(0,))


def _vtake(v, idx1d):
    idx2d = lax.broadcast_in_dim(idx1d, (LANES, 1), (0,))
    return lax.gather(v, idx2d, _GDN, (1,),
                      mode=lax.GatherScatterMode.PROMISE_IN_BOUNDS)


def _encode_body(img_hbm, pose_hbm, table_hbm, res_hbm, out0_hbm, out1_hbm,
                 cbuf_v, dbuf_v, coords_v, resb_v, idx_v, wbuf_v, rows_v,
                 out0_v, out1_v, sem):
    n_pts = img_hbm.shape[0] // 2
    np_per = n_pts // NW
    n_blocks = np_per // PB
    wid = lax.axis_index("s") * NC + lax.axis_index("c")

    pltpu.sync_copy(res_hbm, resb_v)
    iota = lax.iota(jnp.int32, LANES)
    plo2d = lax.shift_right_logical(iota, 1)
    phi2d = plo2d + 8
    par = iota & 1
    pe = lax.shift_left(iota & 7, 1)
    po = pe + 1
    lo_half = iota < 8

    def blk_body(blk, carry):
        base = wid * np_per + blk * PB
        pltpu.sync_copy(img_hbm.at[pl.ds(2 * base, 2 * PB)], cbuf_v)
        pltpu.sync_copy(pose_hbm.at[pl.ds(2 * base, 2 * PB)], dbuf_v)
        for g in range(G):
            for src, d0 in ((cbuf_v, 0), (dbuf_v, 2)):
                a = src[pl.ds(2 * g * LANES, LANES)]
                b2 = src[pl.ds((2 * g + 1) * LANES, LANES)]
                coords_v[d0, pl.ds(g * LANES, LANES)] = jnp.where(
                    lo_half, _vtake(a, pe), _vtake(b2, pe))
                coords_v[d0 + 1, pl.ds(g * LANES, LANES)] = jnp.where(
                    lo_half, _vtake(a, po), _vtake(b2, po))

        def lvl_body(l, carry2):
            resv = resb_v[l]
            lofs2 = l * (2 * T)

            for g in range(G):
                av, w_sel = [], []
                for d in range(4):
                    c = coords_v[d, pl.ds(g * LANES, LANES)]
                    x = c * resv
                    xi = x.astype(jnp.int32)
                    wd = x - xi.astype(jnp.float32)
                    om = jnp.float32(1.0) - wd
                    if d == 0:
                        a0 = xi
                        a1 = xi + 1
                    else:
                        a0 = xi * jnp.int32(_PRIMES[d])
                        a1 = a0 + jnp.int32(_PRIMES[d])
                    av.append((a0, a1))
                    w_sel.append((om, wd))
                w01 = {(b0, b1): w_sel[0][b0] * w_sel[1][b1]
                       for b0 in range(2) for b1 in range(2)}
                w23 = {(b2, b3): w_sel[2][b2] * w_sel[3][b3]
                       for b2 in range(2) for b3 in range(2)}
                half = g // (G // 2)
                col = (2 * g * LANES) % PB
                for c16 in range(16):
                    b = [(c16 >> d) & 1 for d in range(4)]
                    h = av[0][b[0]] ^ av[1][b[1]] ^ av[2][b[2]] ^ av[3][b[3]]
                    h2 = lax.shift_left(h & MASK, 1) + lofs2
                    idx_v[2 * c16 + half, pl.ds(col, LANES)] = _vtake(h2, plo2d) + par
                    idx_v[2 * c16 + half, pl.ds(col + LANES, LANES)] = _vtake(h2, phi2d) + par
                    wt = w01[(b[0], b[1])] * w23[(b[2], b[3])]
                    wbuf_v[c16, pl.ds(2 * g * LANES, LANES)] = _vtake(wt, plo2d)
                    wbuf_v[c16, pl.ds((2 * g + 1) * LANES, LANES)] = _vtake(wt, phi2d)

            cps = [pltpu.async_copy(table_hbm.at[idx_v.at[r]],
                                    rows_v.at[r], sem)
                   for r in range(32)]
            for cp in cps:
                cp.wait()

            for jj in range(G):
                accs = []
                for j in (2 * jj, 2 * jj + 1):
                    rhalf = j // G
                    rcol = (j * LANES) % PB
                    acc = None
                    for c16 in range(16):
                        wt = wbuf_v[c16, pl.ds(j * LANES, LANES)]
                        fv = rows_v[2 * c16 + rhalf, pl.ds(rcol, LANES)]
                        acc = wt * fv if acc is None else acc + wt * fv
                    accs.append(acc)
                a, b2 = accs
                f0 = jnp.where(lo_half, _vtake(a, pe), _vtake(b2, pe))
                f1 = jnp.where(lo_half, _vtake(a, po), _vtake(b2, po))
                out0_v[l, pl.ds(jj * LANES, LANES)] = f0
                out1_v[l, pl.ds(jj * LANES, LANES)] = f1
            return carry2

        lax.fori_loop(0, N_LEVELS, lvl_body, 0)
        for l2 in range(N_LEVELS):
            pltpu.sync_copy(out0_v.at[l2],
                            out0_hbm.at[pl.ds(l2 * n_pts + base, PB)])
            pltpu.sync_copy(out1_v.at[l2],
                            out1_hbm.at[pl.ds(l2 * n_pts + base, PB)])
        return carry

    lax.fori_loop(0, n_blocks, blk_body, 0)


def _encode(img_flat, pose_flat, table, resb):
    n_pts = img_flat.shape[0] // 2
    fn = pl.kernel(
        _encode_body,
        out_type=(jax.ShapeDtypeStruct((N_LEVELS * n_pts,), jnp.float32),
                  jax.ShapeDtypeStruct((N_LEVELS * n_pts,), jnp.float32)),
        mesh=plsc.VectorSubcoreMesh(core_axis_name="c", subcore_axis_name="s",
                                    num_cores=NC, num_subcores=NS),
        scratch_types=[
            pltpu.VMEM((F * PB,), jnp.float32),
            pltpu.VMEM((F * PB,), jnp.float32),
            pltpu.VMEM((4, PB), jnp.float32),
            pltpu.VMEM((N_LEVELS, LANES), jnp.float32),
            pltpu.VMEM((32, PB), jnp.int32),
            pltpu.VMEM((16, F * PB), jnp.float32),
            pltpu.VMEM((32, PB), jnp.float32),
            pltpu.VMEM((N_LEVELS, PB), jnp.float32),
            pltpu.VMEM((N_LEVELS, PB), jnp.float32),
            pltpu.SemaphoreType.DMA,
        ],
    )
    return fn(img_flat, pose_flat, table, resb)


_DN0 = (((0,), (0,)), ((), ()))


def _mlp_body(e0_ref, e1_ref, w1e_ref, w1o_ref, w2_ref, w3_ref, o_ref):
    h = lax.dot_general(e0_ref[...], w1e_ref[...], _DN0,
                        preferred_element_type=jnp.float32)
    h = h + lax.dot_general(e1_ref[...], w1o_ref[...], _DN0,
                            preferred_element_type=jnp.float32)
    h = jnp.maximum(h, 0.0)
    h = jnp.maximum(jnp.dot(h, w2_ref[...], preferred_element_type=jnp.float32), 0.0)
    o_ref[...] = jnp.dot(h, w3_ref[...], preferred_element_type=jnp.float32)


def _mlp(enc0, enc1, W1, W2, W3):
    n_pts = enc0.shape[1]
    nb = 2048
    w1e = W1[0::2]
    w1o = W1[1::2]
    return pl.pallas_call(
        _mlp_body,
        grid=(n_pts // nb,),
        in_specs=[
            pl.BlockSpec((N_LEVELS, nb), lambda i: (0, i)),
            pl.BlockSpec((N_LEVELS, nb), lambda i: (0, i)),
            pl.BlockSpec((N_LEVELS, 64), lambda i: (0, 0)),
            pl.BlockSpec((N_LEVELS, 64), lambda i: (0, 0)),
            pl.BlockSpec((64, 64), lambda i: (0, 0)),
            pl.BlockSpec((64, 3), lambda i: (0, 0)),
        ],
        out_specs=pl.BlockSpec((nb, 3), lambda i: (i, 0)),
        out_shape=jax.ShapeDtypeStruct((n_pts, 3), jnp.float32),
    )(enc0, enc1, w1e, w1o, W2, W3)


def kernel(img_pixel_indices, pose_extended, table, W1, W2, W3):
    n_pts = img_pixel_indices.shape[0]
    resb = jnp.tile(jnp.asarray(_RES, jnp.float32)[:, None], (1, LANES))
    enc0, enc1 = _encode(img_pixel_indices.reshape(-1),
                         pose_extended.reshape(-1),
                         table.reshape(-1), resb)
    enc0 = enc0.reshape(N_LEVELS, n_pts)
    enc1 = enc1.reshape(N_LEVELS, n_pts)
    return _mlp(enc0, enc1, W1, W2, W3)


# speedup vs baseline: 2.4544x; 2.4544x over previous
import numpy as np
import jax
import jax.numpy as jnp
from jax import lax
from jax.experimental import pallas as pl
from jax.experimental.pallas import tpu as pltpu
from jax.experimental.pallas import tpu_sc as plsc

N_LEVELS = 16
F = 2
LOG2_T = 19
T = 1 << LOG2_T
MASK = T - 1
BASE_RES = 16
PLS = 1.39

LANES = 16
NC, NS = 2, 16
NW = NC * NS
PB = 128
G = PB // LANES

_RES = [float(np.floor(BASE_RES * (PLS ** l))) for l in range(N_LEVELS)]
_PRIMES_U = [1, 2654435761, 805459861, 3674653429]
_PRIMES = [p - (1 << 32) if p >= (1 << 31) else p for p in _PRIMES_U]

_GDN = lax.GatherDimensionNumbers(
    offset_dims=(), collapsed_slice_dims=(0,), start_index_map=(0,))


def _vtake(v, idx1d):
    idx2d = lax.broadcast_in_dim(idx1d, (LANES, 1), (0,))
    return lax.gather(v, idx2d, _GDN, (1,),
                      mode=lax.GatherScatterMode.PROMISE_IN_BOUNDS)


def _encode_body(img_hbm, pose_hbm, table_hbm, res_hbm, out0_hbm, out1_hbm,
                 coords_v, resb_v, idx_v, wbuf_v, rows_v,
                 out0_v, out1_v, sem):
    n_pts = img_hbm.shape[0] * PB
    np_per = n_pts // NW
    n_blocks = np_per // PB
    wid = lax.axis_index("s") * NC + lax.axis_index("c")

    pltpu.sync_copy(res_hbm, resb_v)
    iota = lax.iota(jnp.int32, LANES)
    plo2d = lax.shift_right_logical(iota, 1)
    phi2d = plo2d + 8
    par128 = lax.shift_left(iota & 1, 7)
    pe = lax.shift_left(iota & 7, 1)
    po = pe + 1
    lo_half = iota < 8

    def blk_body(blk, carry):
        base = wid * np_per + blk * PB
        gblk = base // PB
        pltpu.sync_copy(img_hbm.at[gblk], coords_v.at[pl.ds(0, 2)])
        pltpu.sync_copy(pose_hbm.at[gblk], coords_v.at[pl.ds(2, 2)])

        def lvl_body(l, carry2):
            resv = resb_v[l]
            lofs2 = l * (2 * T)

            for g in range(G):
                av, w_sel = [], []
                for d in range(4):
                    c = coords_v[d, pl.ds(g * LANES, LANES)]
                    x = c * resv
                    xi = x.astype(jnp.int32)
                    wd = x - xi.astype(jnp.float32)
                    om = jnp.float32(1.0) - wd
                    if d == 0:
                        a0 = xi
                        a1 = xi + 1
                    else:
                        a0 = xi * jnp.int32(_PRIMES[d])
                        a1 = a0 + jnp.int32(_PRIMES[d])
                    av.append((a0, a1))
                    w_sel.append((om, wd))
                w01 = {(b0, b1): w_sel[0][b0] * w_sel[1][b1]
                       for b0 in range(2) for b1 in range(2)}
                w23 = {(b2, b3): w_sel[2][b2] * w_sel[3][b3]
                       for b2 in range(2) for b3 in range(2)}
                half = g // (G // 2)
                col = (2 * g * LANES) % PB
                for c16 in range(16):
                    b = [(c16 >> d) & 1 for d in range(4)]
                    h = av[0][b[0]] ^ av[1][b[1]] ^ av[2][b[2]] ^ av[3][b[3]]
                    hm = h & MASK
                    h2 = lax.shift_left(hm, 1) + lofs2 - (hm & 127)
                    idx_v[2 * c16 + half, pl.ds(col, LANES)] = _vtake(h2, plo2d) + par128
                    idx_v[2 * c16 + half, pl.ds(col + LANES, LANES)] = _vtake(h2, phi2d) + par128
                    wt = w01[(b[0], b[1])] * w23[(b[2], b[3])]
                    wbuf_v[c16, pl.ds(2 * g * LANES, LANES)] = _vtake(wt, plo2d)
                    wbuf_v[c16, pl.ds((2 * g + 1) * LANES, LANES)] = _vtake(wt, phi2d)

            cps = [pltpu.async_copy(table_hbm.at[idx_v.at[r]],
                                    rows_v.at[r], sem)
                   for r in range(32)]
            for cp in cps:
                cp.wait()

            for jj in range(G):
                accs = []
                for j in (2 * jj, 2 * jj + 1):
                    rhalf = j // G
                    rcol = (j * LANES) % PB
                    acc = None
                    for c16 in range(16):
                        wt = wbuf_v[c16, pl.ds(j * LANES, LANES)]
                        fv = rows_v[2 * c16 + rhalf, pl.ds(rcol, LANES)]
                        acc = wt * fv if acc is None else acc + wt * fv
                    accs.append(acc)
                a, b2 = accs
                f0 = jnp.where(lo_half, _vtake(a, pe), _vtake(b2, pe))
                f1 = jnp.where(lo_half, _vtake(a, po), _vtake(b2, po))
                out0_v[l, pl.ds(jj * LANES, LANES)] = f0
                out1_v[l, pl.ds(jj * LANES, LANES)] = f1
            return carry2

        lax.fori_loop(0, N_LEVELS, lvl_body, 0)
        for l2 in range(N_LEVELS):
            pltpu.sync_copy(out0_v.at[l2],
                            out0_hbm.at[pl.ds(l2 * n_pts + base, PB)])
            pltpu.sync_copy(out1_v.at[l2],
                            out1_hbm.at[pl.ds(l2 * n_pts + base, PB)])
        return carry

    lax.fori_loop(0, n_blocks, blk_body, 0)


def _encode(img_n, pose_n, table, resb):
    n_pts = img_n.shape[0] * PB
    fn = pl.kernel(
        _encode_body,
        out_type=(jax.ShapeDtypeStruct((N_LEVELS * n_pts,), jnp.float32),
                  jax.ShapeDtypeStruct((N_LEVELS * n_pts,), jnp.float32)),
        mesh=plsc.VectorSubcoreMesh(core_axis_name="c", subcore_axis_name="s",
                                    num_cores=NC, num_subcores=NS),
        scratch_types=[
            pltpu.VMEM((4, PB), jnp.float32),
            pltpu.VMEM((N_LEVELS, LANES), jnp.float32),
            pltpu.VMEM((32, PB), jnp.int32),
            pltpu.VMEM((16, F * PB), jnp.float32),
            pltpu.VMEM((32, PB), jnp.float32),
            pltpu.VMEM((N_LEVELS, PB), jnp.float32),
            pltpu.VMEM((N_LEVELS, PB), jnp.float32),
            pltpu.SemaphoreType.DMA,
        ],
    )
    return fn(img_n, pose_n, table, resb)


_DN0 = (((0,), (0,)), ((), ()))


def _mlp_body(e0_ref, e1_ref, w1e_ref, w1o_ref, w2_ref, w3_ref, o_ref):
    h = lax.dot_general(e0_ref[...], w1e_ref[...], _DN0,
                        preferred_element_type=jnp.float32)
    h = h + lax.dot_general(e1_ref[...], w1o_ref[...], _DN0,
                            preferred_element_type=jnp.float32)
    h = jnp.maximum(h, 0.0)
    h = jnp.maximum(jnp.dot(h, w2_ref[...], preferred_element_type=jnp.float32), 0.0)
    o_ref[...] = jnp.dot(h, w3_ref[...], preferred_element_type=jnp.float32)


def _mlp(enc0, enc1, W1, W2, W3):
    n_pts = enc0.shape[1]
    nb = 2048
    w1e = W1[0::2]
    w1o = W1[1::2]
    return pl.pallas_call(
        _mlp_body,
        grid=(n_pts // nb,),
        in_specs=[
            pl.BlockSpec((N_LEVELS, nb), lambda i: (0, i)),
            pl.BlockSpec((N_LEVELS, nb), lambda i: (0, i)),
            pl.BlockSpec((N_LEVELS, 64), lambda i: (0, 0)),
            pl.BlockSpec((N_LEVELS, 64), lambda i: (0, 0)),
            pl.BlockSpec((64, 64), lambda i: (0, 0)),
            pl.BlockSpec((64, 3), lambda i: (0, 0)),
        ],
        out_specs=pl.BlockSpec((nb, 3), lambda i: (i, 0)),
        out_shape=jax.ShapeDtypeStruct((n_pts, 3), jnp.float32),
    )(enc0, enc1, w1e, w1o, W2, W3)


def kernel(img_pixel_indices, pose_extended, table, W1, W2, W3):
    n_pts = img_pixel_indices.shape[0]
    resb = jnp.tile(jnp.asarray(_RES, jnp.float32)[:, None], (1, LANES))
    tab_n = table.reshape(-1, PB, F).transpose(0, 2, 1).reshape(-1)
    img_n = img_pixel_indices.reshape(-1, PB, F).transpose(0, 2, 1)
    pose_n = pose_extended.reshape(-1, PB, F).transpose(0, 2, 1)
    enc0, enc1 = _encode(img_n, pose_n, tab_n, resb)
    enc0 = enc0.reshape(N_LEVELS, n_pts)
    enc1 = enc1.reshape(N_LEVELS, n_pts)
    return _mlp(enc0, enc1, W1, W2, W3)


# speedup vs baseline: 2.6837x; 1.0934x over previous
import numpy as np
import jax
import jax.numpy as jnp
from jax import lax
from jax.experimental import pallas as pl
from jax.experimental.pallas import tpu as pltpu
from jax.experimental.pallas import tpu_sc as plsc

N_LEVELS = 16
F = 2
LOG2_T = 19
T = 1 << LOG2_T
MASK = T - 1
BASE_RES = 16
PLS = 1.39

LANES = 16
NC, NS = 2, 16
NW = NC * NS
PB = 128
G = PB // LANES

_RES = [float(np.floor(BASE_RES * (PLS ** l))) for l in range(N_LEVELS)]
_PRIMES_U = [1, 2654435761, 805459861, 3674653429]
_PRIMES = [p - (1 << 32) if p >= (1 << 31) else p for p in _PRIMES_U]

_GDN = lax.GatherDimensionNumbers(
    offset_dims=(), collapsed_slice_dims=(0,), start_index_map=(0,))


def _vtake(v, idx1d):
    idx2d = lax.broadcast_in_dim(idx1d, (LANES, 1), (0,))
    return lax.gather(v, idx2d, _GDN, (1,),
                      mode=lax.GatherScatterMode.PROMISE_IN_BOUNDS)


def _encode_body(img_hbm, pose_hbm, table_hbm, res_hbm, out0_hbm, out1_hbm,
                 coords_v, resb_v, idx_v, wbuf_v, rows_v,
                 out0_v, out1_v, sem):
    n_pts = img_hbm.shape[0] * PB
    np_per = n_pts // NW
    n_blocks = np_per // PB
    wid = lax.axis_index("s") * NC + lax.axis_index("c")

    pltpu.sync_copy(res_hbm, resb_v)
    iota = lax.iota(jnp.int32, LANES)
    plo2d = lax.shift_right_logical(iota, 1)
    phi2d = plo2d + 8
    par128 = lax.shift_left(iota & 1, 7)
    pe = lax.shift_left(iota & 7, 1)
    po = pe + 1
    lo_half = iota < 8

    def pass_a(l, slot):
        resv = resb_v[l]
        lofs2 = l * (2 * T)
        for g in range(G):
            av, w_sel = [], []
            for d in range(4):
                c = coords_v[d, pl.ds(g * LANES, LANES)]
                x = c * resv
                xi = x.astype(jnp.int32)
                wd = x - xi.astype(jnp.float32)
                om = jnp.float32(1.0) - wd
                if d == 0:
                    a0 = xi
                    a1 = xi + 1
                else:
                    a0 = xi * jnp.int32(_PRIMES[d])
                    a1 = a0 + jnp.int32(_PRIMES[d])
                av.append((a0, a1))
                w_sel.append((om, wd))
            w01 = {(b0, b1): w_sel[0][b0] * w_sel[1][b1]
                   for b0 in range(2) for b1 in range(2)}
            w23 = {(b2, b3): w_sel[2][b2] * w_sel[3][b3]
                   for b2 in range(2) for b3 in range(2)}
            half = g // (G // 2)
            col = (2 * g * LANES) % PB
            for c16 in range(16):
                b = [(c16 >> d) & 1 for d in range(4)]
                h = av[0][b[0]] ^ av[1][b[1]] ^ av[2][b[2]] ^ av[3][b[3]]
                hm = h & MASK
                h2 = lax.shift_left(hm, 1) + lofs2 - (hm & 127)
                idx_v[slot, 2 * c16 + half, pl.ds(col, LANES)] = _vtake(h2, plo2d) + par128
                idx_v[slot, 2 * c16 + half, pl.ds(col + LANES, LANES)] = _vtake(h2, phi2d) + par128
                wt = w01[(b[0], b[1])] * w23[(b[2], b[3])]
                wbuf_v[slot, c16, pl.ds(2 * g * LANES, LANES)] = _vtake(wt, plo2d)
                wbuf_v[slot, c16, pl.ds((2 * g + 1) * LANES, LANES)] = _vtake(wt, phi2d)

    def pass_b(l, slot):
        for jj in range(G):
            accs = []
            for j in (2 * jj, 2 * jj + 1):
                rhalf = j // G
                rcol = (j * LANES) % PB
                acc = None
                for c16 in range(16):
                    wt = wbuf_v[slot, c16, pl.ds(j * LANES, LANES)]
                    fv = rows_v[slot, 2 * c16 + rhalf, pl.ds(rcol, LANES)]
                    acc = wt * fv if acc is None else acc + wt * fv
                accs.append(acc)
            a, b2 = accs
            f0 = jnp.where(lo_half, _vtake(a, pe), _vtake(b2, pe))
            f1 = jnp.where(lo_half, _vtake(a, po), _vtake(b2, po))
            out0_v[l, pl.ds(jj * LANES, LANES)] = f0
            out1_v[l, pl.ds(jj * LANES, LANES)] = f1

    def blk_body(blk, carry):
        base = wid * np_per + blk * PB
        gblk = base // PB
        pltpu.sync_copy(img_hbm.at[gblk], coords_v.at[pl.ds(0, 2)])
        pltpu.sync_copy(pose_hbm.at[gblk], coords_v.at[pl.ds(2, 2)])

        pass_a(0, 0)

        def lvl_body(l, carry2):
            p = l & 1
            cps = [pltpu.async_copy(table_hbm.at[idx_v.at[p, r]],
                                    rows_v.at[p, r], sem)
                   for r in range(32)]

            @pl.when(l < N_LEVELS - 1)
            def _():
                pass_a(l + 1, 1 - p)

            for cp in cps:
                cp.wait()
            pass_b(l, p)
            return carry2

        lax.fori_loop(0, N_LEVELS, lvl_body, 0)
        for l2 in range(N_LEVELS):
            pltpu.sync_copy(out0_v.at[l2],
                            out0_hbm.at[pl.ds(l2 * n_pts + base, PB)])
            pltpu.sync_copy(out1_v.at[l2],
                            out1_hbm.at[pl.ds(l2 * n_pts + base, PB)])
        return carry

    lax.fori_loop(0, n_blocks, blk_body, 0)


def _encode(img_n, pose_n, table, resb):
    n_pts = img_n.shape[0] * PB
    fn = pl.kernel(
        _encode_body,
        out_type=(jax.ShapeDtypeStruct((N_LEVELS * n_pts,), jnp.float32),
                  jax.ShapeDtypeStruct((N_LEVELS * n_pts,), jnp.float32)),
        mesh=plsc.VectorSubcoreMesh(core_axis_name="c", subcore_axis_name="s",
                                    num_cores=NC, num_subcores=NS),
        scratch_types=[
            pltpu.VMEM((4, PB), jnp.float32),
            pltpu.VMEM((N_LEVELS, LANES), jnp.float32),
            pltpu.VMEM((2, 32, PB), jnp.int32),
            pltpu.VMEM((2, 16, F * PB), jnp.float32),
            pltpu.VMEM((2, 32, PB), jnp.float32),
            pltpu.VMEM((N_LEVELS, PB), jnp.float32),
            pltpu.VMEM((N_LEVELS, PB), jnp.float32),
            pltpu.SemaphoreType.DMA,
        ],
    )
    return fn(img_n, pose_n, table, resb)


_DN0 = (((0,), (0,)), ((), ()))


def _mlp_body(e0_ref, e1_ref, w1e_ref, w1o_ref, w2_ref, w3_ref, o_ref):
    h = lax.dot_general(e0_ref[...], w1e_ref[...], _DN0,
                        preferred_element_type=jnp.float32)
    h = h + lax.dot_general(e1_ref[...], w1o_ref[...], _DN0,
                            preferred_element_type=jnp.float32)
    h = jnp.maximum(h, 0.0)
    h = jnp.maximum(jnp.dot(h, w2_ref[...], preferred_element_type=jnp.float32), 0.0)
    o_ref[...] = jnp.dot(h, w3_ref[...], preferred_element_type=jnp.float32)


def _mlp(enc0, enc1, W1, W2, W3):
    n_pts = enc0.shape[1]
    nb = 2048
    w1e = W1[0::2]
    w1o = W1[1::2]
    return pl.pallas_call(
        _mlp_body,
        grid=(n_pts // nb,),
        in_specs=[
            pl.BlockSpec((N_LEVELS, nb), lambda i: (0, i)),
            pl.BlockSpec((N_LEVELS, nb), lambda i: (0, i)),
            pl.BlockSpec((N_LEVELS, 64), lambda i: (0, 0)),
            pl.BlockSpec((N_LEVELS, 64), lambda i: (0, 0)),
            pl.BlockSpec((64, 64), lambda i: (0, 0)),
            pl.BlockSpec((64, 3), lambda i: (0, 0)),
        ],
        out_specs=pl.BlockSpec((nb, 3), lambda i: (i, 0)),
        out_shape=jax.ShapeDtypeStruct((n_pts, 3), jnp.float32),
    )(enc0, enc1, w1e, w1o, W2, W3)


def kernel(img_pixel_indices, pose_extended, table, W1, W2, W3):
    n_pts = img_pixel_indices.shape[0]
    resb = jnp.tile(jnp.asarray(_RES, jnp.float32)[:, None], (1, LANES))
    tab_n = table.reshape(-1, PB, F).transpose(0, 2, 1).reshape(-1)
    img_n = img_pixel_indices.reshape(-1, PB, F).transpose(0, 2, 1)
    pose_n = pose_extended.reshape(-1, PB, F).transpose(0, 2, 1)
    enc0, enc1 = _encode(img_n, pose_n, tab_n, resb)
    enc0 = enc0.reshape(N_LEVELS, n_pts)
    enc1 = enc1.reshape(N_LEVELS, n_pts)
    return _mlp(enc0, enc1, W1, W2, W3)


# speedup vs baseline: 2.8877x; 1.0760x over previous
import numpy as np
import jax
import jax.numpy as jnp
from jax import lax
from jax.experimental import pallas as pl
from jax.experimental.pallas import tpu as pltpu
from jax.experimental.pallas import tpu_sc as plsc

N_LEVELS = 16
F = 2
LOG2_T = 19
T = 1 << LOG2_T
MASK = T - 1
BASE_RES = 16
PLS = 1.39

LANES = 16
NC, NS = 2, 16
NW = NC * NS
PB = 128
G = PB // LANES

_RES = [float(np.floor(BASE_RES * (PLS ** l))) for l in range(N_LEVELS)]
_PRIMES_U = [1, 2654435761, 805459861, 3674653429]
_PRIMES = [p - (1 << 32) if p >= (1 << 31) else p for p in _PRIMES_U]

_GDN = lax.GatherDimensionNumbers(
    offset_dims=(), collapsed_slice_dims=(0,), start_index_map=(0,))


def _vtake(v, idx1d):
    idx2d = lax.broadcast_in_dim(idx1d, (LANES, 1), (0,))
    return lax.gather(v, idx2d, _GDN, (1,),
                      mode=lax.GatherScatterMode.PROMISE_IN_BOUNDS)


def _encode_body(img_hbm, pose_hbm, table_hbm, res_hbm, out0_hbm, out1_hbm,
                 coords_v, resb_v, idx_v, wbuf_v, rows_v,
                 out0_v, out1_v, sem):
    n_pts = img_hbm.shape[0] * PB
    np_per = n_pts // NW
    n_blocks = np_per // PB
    wid = lax.axis_index("s") * NC + lax.axis_index("c")

    pltpu.sync_copy(res_hbm, resb_v)
    iota = lax.iota(jnp.int32, LANES)
    plo2d = lax.shift_right_logical(iota, 1)
    phi2d = plo2d + 8
    par128 = lax.shift_left(iota & 1, 7)
    pe = lax.shift_left(iota & 7, 1)
    po = pe + 1
    lo_half = iota < 8

    def pass_a(l, slot, wslot):
        resv = resb_v[l]
        lofs2 = l * (2 * T)
        for g in range(G):
            av, w_sel = [], []
            for d in range(4):
                c = coords_v[d, pl.ds(g * LANES, LANES)]
                x = c * resv
                xi = x.astype(jnp.int32)
                wd = x - xi.astype(jnp.float32)
                om = jnp.float32(1.0) - wd
                if d == 0:
                    a0 = xi
                    a1 = xi + 1
                else:
                    a0 = xi * jnp.int32(_PRIMES[d])
                    a1 = a0 + jnp.int32(_PRIMES[d])
                av.append((a0, a1))
                w_sel.append((om, wd))
            w01 = {(b0, b1): w_sel[0][b0] * w_sel[1][b1]
                   for b0 in range(2) for b1 in range(2)}
            w23 = {(b2, b3): w_sel[2][b2] * w_sel[3][b3]
                   for b2 in range(2) for b3 in range(2)}
            half = g // (G // 2)
            col = (2 * g * LANES) % PB
            for c16 in range(16):
                b = [(c16 >> d) & 1 for d in range(4)]
                h = av[0][b[0]] ^ av[1][b[1]] ^ av[2][b[2]] ^ av[3][b[3]]
                hm = h & MASK
                h2 = lax.shift_left(hm, 1) + lofs2 - (hm & 127)
                idx_v[slot, 2 * c16 + half, pl.ds(col, LANES)] = _vtake(h2, plo2d) + par128
                idx_v[slot, 2 * c16 + half, pl.ds(col + LANES, LANES)] = _vtake(h2, phi2d) + par128
                wt = w01[(b[0], b[1])] * w23[(b[2], b[3])]
                wbuf_v[wslot, c16, pl.ds(2 * g * LANES, LANES)] = _vtake(wt, plo2d)
                wbuf_v[wslot, c16, pl.ds((2 * g + 1) * LANES, LANES)] = _vtake(wt, phi2d)

    def pass_b(l, slot, wslot):
        for jj in range(G):
            accs = []
            for j in (2 * jj, 2 * jj + 1):
                rhalf = j // G
                rcol = (j * LANES) % PB
                acc = None
                for c16 in range(16):
                    wt = wbuf_v[wslot, c16, pl.ds(j * LANES, LANES)]
                    fv = rows_v[slot, 2 * c16 + rhalf, pl.ds(rcol, LANES)]
                    acc = wt * fv if acc is None else acc + wt * fv
                accs.append(acc)
            a, b2 = accs
            f0 = jnp.where(lo_half, _vtake(a, pe), _vtake(b2, pe))
            f1 = jnp.where(lo_half, _vtake(a, po), _vtake(b2, po))
            out0_v[l, pl.ds(jj * LANES, LANES)] = f0
            out1_v[l, pl.ds(jj * LANES, LANES)] = f1

    def blk_body(blk, carry):
        base = wid * np_per + blk * PB
        gblk = base // PB
        pltpu.sync_copy(img_hbm.at[gblk], coords_v.at[pl.ds(0, 2)])
        pltpu.sync_copy(pose_hbm.at[gblk], coords_v.at[pl.ds(2, 2)])

        pass_a(0, 0, 0)

        def lvl_body(l, carry2):
            p = l & 1
            cps = [pltpu.async_copy(table_hbm.at[idx_v.at[p, r]],
                                    rows_v.at[p, r], sem)
                   for r in range(32)]

            @pl.when(l < N_LEVELS - 1)
            def _():
                pass_a(l + 1, 1 - p, (l + 1) & 3)

            @pl.when(l > 0)
            def _():
                pass_b(l - 1, 1 - p, (l - 1) & 3)

            for cp in cps:
                cp.wait()
            return carry2

        lax.fori_loop(0, N_LEVELS, lvl_body, 0)
        pass_b(N_LEVELS - 1, (N_LEVELS - 1) & 1, (N_LEVELS - 1) & 3)
        for l2 in range(N_LEVELS):
            pltpu.sync_copy(out0_v.at[l2],
                            out0_hbm.at[pl.ds(l2 * n_pts + base, PB)])
            pltpu.sync_copy(out1_v.at[l2],
                            out1_hbm.at[pl.ds(l2 * n_pts + base, PB)])
        return carry

    lax.fori_loop(0, n_blocks, blk_body, 0)


def _encode(img_n, pose_n, table, resb):
    n_pts = img_n.shape[0] * PB
    fn = pl.kernel(
        _encode_body,
        out_type=(jax.ShapeDtypeStruct((N_LEVELS * n_pts,), jnp.float32),
                  jax.ShapeDtypeStruct((N_LEVELS * n_pts,), jnp.float32)),
        mesh=plsc.VectorSubcoreMesh(core_axis_name="c", subcore_axis_name="s",
                                    num_cores=NC, num_subcores=NS),
        scratch_types=[
            pltpu.VMEM((4, PB), jnp.float32),
            pltpu.VMEM((N_LEVELS, LANES), jnp.float32),
            pltpu.VMEM((2, 32, PB), jnp.int32),
            pltpu.VMEM((4, 16, F * PB), jnp.float32),
            pltpu.VMEM((2, 32, PB), jnp.float32),
            pltpu.VMEM((N_LEVELS, PB), jnp.float32),
            pltpu.VMEM((N_LEVELS, PB), jnp.float32),
            pltpu.SemaphoreType.DMA,
        ],
    )
    return fn(img_n, pose_n, table, resb)


_DN0 = (((0,), (0,)), ((), ()))


def _mlp_body(e0_ref, e1_ref, w1e_ref, w1o_ref, w2_ref, w3_ref, o_ref):
    h = lax.dot_general(e0_ref[...], w1e_ref[...], _DN0,
                        preferred_element_type=jnp.float32)
    h = h + lax.dot_general(e1_ref[...], w1o_ref[...], _DN0,
                            preferred_element_type=jnp.float32)
    h = jnp.maximum(h, 0.0)
    h = jnp.maximum(jnp.dot(h, w2_ref[...], preferred_element_type=jnp.float32), 0.0)
    o_ref[...] = jnp.dot(h, w3_ref[...], preferred_element_type=jnp.float32)


def _mlp(enc0, enc1, W1, W2, W3):
    n_pts = enc0.shape[1]
    nb = 2048
    w1e = W1[0::2]
    w1o = W1[1::2]
    return pl.pallas_call(
        _mlp_body,
        grid=(n_pts // nb,),
        in_specs=[
            pl.BlockSpec((N_LEVELS, nb), lambda i: (0, i)),
            pl.BlockSpec((N_LEVELS, nb), lambda i: (0, i)),
            pl.BlockSpec((N_LEVELS, 64), lambda i: (0, 0)),
            pl.BlockSpec((N_LEVELS, 64), lambda i: (0, 0)),
            pl.BlockSpec((64, 64), lambda i: (0, 0)),
            pl.BlockSpec((64, 3), lambda i: (0, 0)),
        ],
        out_specs=pl.BlockSpec((nb, 3), lambda i: (i, 0)),
        out_shape=jax.ShapeDtypeStruct((n_pts, 3), jnp.float32),
    )(enc0, enc1, w1e, w1o, W2, W3)


def kernel(img_pixel_indices, pose_extended, table, W1, W2, W3):
    n_pts = img_pixel_indices.shape[0]
    resb = jnp.tile(jnp.asarray(_RES, jnp.float32)[:, None], (1, LANES))
    tab_n = table.reshape(-1, PB, F).transpose(0, 2, 1).reshape(-1)
    img_n = img_pixel_indices.reshape(-1, PB, F).transpose(0, 2, 1)
    pose_n = pose_extended.reshape(-1, PB, F).transpose(0, 2, 1)
    enc0, enc1 = _encode(img_n, pose_n, tab_n, resb)
    enc0 = enc0.reshape(N_LEVELS, n_pts)
    enc1 = enc1.reshape(N_LEVELS, n_pts)
    return _mlp(enc0, enc1, W1, W2, W3)
